# baseline (speedup 1.0000x reference)
import jax
import jax.numpy as jnp
from jax import lax
from jax.experimental import pallas as pl
from jax.experimental.pallas import tpu as pltpu
from jax.experimental.pallas import tpu_sc as plsc

B = 16384
K = 32
NC = 2
NS = 16
NW = NC * NS
BPW = B // NW
CHUNK = 128
NCH = BPW // CHUNK
L = 16


def _body(ut, it, uid, iid, out_hbm,
          idx_u, idx_i, u_buf, i_buf, out_v, sem):
    wid = lax.axis_index("s") * NC + lax.axis_index("c")
    base = wid * BPW

    pltpu.sync_copy(uid.at[wid], idx_u)
    pltpu.sync_copy(iid.at[wid], idx_i)

    def row(k, _):
        copies = []
        for j in range(NCH):
            dst = pl.ds(j * CHUNK, CHUNK)
            copies.append(pltpu.async_copy(
                ut.at[k].at[idx_u.at[j]], u_buf.at[k].at[dst], sem))
            copies.append(pltpu.async_copy(
                it.at[k].at[idx_i.at[j]], i_buf.at[k].at[dst], sem))
        for c in copies:
            c.wait()
        return 0

    lax.fori_loop(0, K, row, 0)

    def blk(i, _):
        b0 = pl.multiple_of(i * L, L)
        acc = jnp.zeros((L,), jnp.float32)
        for k in range(K):
            acc = acc + u_buf[k, pl.ds(b0, L)] * i_buf[k, pl.ds(b0, L)]
        out_v[pl.ds(b0, L)] = acc
        return 0

    lax.fori_loop(0, BPW // L, blk, 0)

    pltpu.sync_copy(out_v, out_hbm.at[pl.ds(base, BPW)])


def kernel(user_id, item_id, user_table, item_table):
    ut = user_table.T
    it = item_table.T
    uid = user_id.astype(jnp.int32).reshape(NW, NCH, CHUNK)
    iid = item_id.astype(jnp.int32).reshape(NW, NCH, CHUNK)
    mesh = plsc.VectorSubcoreMesh(core_axis_name="c", subcore_axis_name="s",
                                  num_cores=NC, num_subcores=NS)
    out = pl.kernel(
        _body,
        out_type=jax.ShapeDtypeStruct((B,), jnp.float32),
        mesh=mesh,
        scratch_types=[
            pltpu.VMEM((NCH, CHUNK), jnp.int32),
            pltpu.VMEM((NCH, CHUNK), jnp.int32),
            pltpu.VMEM((K, BPW), jnp.float32),
            pltpu.VMEM((K, BPW), jnp.float32),
            pltpu.VMEM((BPW,), jnp.float32),
            pltpu.SemaphoreType.DMA,
        ],
        compiler_params=pltpu.CompilerParams(needs_layout_passes=False,
                                             use_tc_tiling_on_sc=False),
    )(ut, it, uid, iid)
    return out.reshape(B, 1)


# speedup vs baseline: 12.1153x; 12.1153x over previous
import jax
import jax.numpy as jnp
from jax import lax
from jax.experimental import pallas as pl
from jax.experimental.pallas import tpu as pltpu
from jax.experimental.pallas import tpu_sc as plsc

B = 16384
K = 32
N = 1000000
G = 8
NC = 2
NS = 16
NW = NC * NS
BPW = B // NW
L = 16
W = 16


def _body(ut, it, uid, iid, out_hbm,
          idx_u_s, idx_i_s,
          stag_u, stag_i, u_flat, i_flat, out_v, sem):
    wid = lax.axis_index("s") * NC + lax.axis_index("c")
    base = wid * BPW

    pltpu.sync_copy(uid.at[pl.ds(base, BPW)], idx_u_s)
    pltpu.sync_copy(iid.at[pl.ds(base, BPW)], idx_i_s)

    def wave(w, _):
        b0 = pl.multiple_of(w * W, W)
        iv_u = idx_u_s[pl.ds(b0, W)]
        iv_i = idx_i_s[pl.ds(b0, W)]
        copies = []
        for t in range(W):
            gu = iv_u[t] >> 3
            gi = iv_i[t] >> 3
            copies.append(pltpu.async_copy(
                ut.at[gu], stag_u.at[pl.ds(t * G, G)], sem))
            copies.append(pltpu.async_copy(
                it.at[gi], stag_i.at[pl.ds(t * G, G)], sem))
        for c in copies:
            c.wait()
        for t in range(W):
            b = b0 + t
            ru = iv_u[t] & 7
            ri = iv_i[t] & 7
            u_flat[pl.ds(b * K, L)] = stag_u[t * G + ru, pl.ds(0, L)]
            u_flat[pl.ds(b * K + L, L)] = stag_u[t * G + ru, pl.ds(L, L)]
            i_flat[pl.ds(b * K, L)] = stag_i[t * G + ri, pl.ds(0, L)]
            i_flat[pl.ds(b * K + L, L)] = stag_i[t * G + ri, pl.ds(L, L)]
        return 0

    lax.fori_loop(0, BPW // W, wave, 0)

    def blk(i, _):
        b0 = pl.multiple_of(i * L, L)
        flat0 = b0 * K + lax.iota(jnp.int32, L) * K
        acc = jnp.zeros((L,), jnp.float32)
        for k in range(K):
            u = plsc.load_gather(u_flat, [flat0 + k])
            v = plsc.load_gather(i_flat, [flat0 + k])
            acc = acc + u * v
        out_v[pl.ds(b0, L)] = acc
        return 0

    lax.fori_loop(0, BPW // L, blk, 0)

    pltpu.sync_copy(out_v, out_hbm.at[pl.ds(base, BPW)])


def kernel(user_id, item_id, user_table, item_table):
    ut = user_table.reshape(N // G, G, K)
    it = item_table.reshape(N // G, G, K)
    uid = user_id.astype(jnp.int32)
    iid = item_id.astype(jnp.int32)
    mesh = plsc.VectorSubcoreMesh(core_axis_name="c", subcore_axis_name="s",
                                  num_cores=NC, num_subcores=NS)
    out = pl.kernel(
        _body,
        out_type=jax.ShapeDtypeStruct((B,), jnp.float32),
        mesh=mesh,
        scratch_types=[
            pltpu.VMEM((BPW,), jnp.int32),
            pltpu.VMEM((BPW,), jnp.int32),
            pltpu.VMEM((W * G, K), jnp.float32),
            pltpu.VMEM((W * G, K), jnp.float32),
            pltpu.VMEM((BPW * K,), jnp.float32),
            pltpu.VMEM((BPW * K,), jnp.float32),
            pltpu.VMEM((BPW,), jnp.float32),
            pltpu.SemaphoreType.DMA,
        ],
        compiler_params=pltpu.CompilerParams(needs_layout_passes=False,
                                             use_tc_tiling_on_sc=True),
    )(ut, it, uid, iid)
    return out.reshape(B, 1)
